# Initial kernel scaffold; baseline (speedup 1.0000x reference)
import jax
import jax.numpy as jnp
from jax.experimental import pallas as pl


def kernel(X_batch, emb, W1, b1, W2, b2, W3, b3):
    raise NotImplementedError("write your pallas kernel here")


# speedup vs baseline: 4.5502x; 4.5502x over previous
import functools

import jax
import jax.numpy as jnp
from jax import lax
from jax.experimental import pallas as pl
from jax.experimental.pallas import tpu as pltpu
from jax.experimental.pallas import tpu_sc as plsc

VOCAB = 100000
EMBED = 64
N_CLASSES = 1000
B = 16384
L = 20

NW = 32
BAGS_PER_W = B // NW
CHUNK_BAGS = 4
CHUNK_IDX = CHUNK_BAGS * L
N_CHUNKS = BAGS_PER_W // CHUNK_BAGS

@functools.cache
def _get_bag_mean_sc():
    mesh = plsc.VectorSubcoreMesh(core_axis_name="c", subcore_axis_name="s")

    @functools.partial(
        pl.kernel,
        out_type=jax.ShapeDtypeStruct((B, EMBED), jnp.float32),
        mesh=mesh,
        scratch_types=[
            pltpu.VMEM((BAGS_PER_W * L,), jnp.int32),
            pltpu.VMEM((CHUNK_IDX, EMBED), jnp.float32),
            pltpu.VMEM((BAGS_PER_W, EMBED), jnp.float32),
            pltpu.SemaphoreType.DMA,
        ],
        compiler_params=pltpu.CompilerParams(use_tc_tiling_on_sc=False),
    )
    def _bag_mean_sc(x_hbm, emb_hbm, out_hbm, idx_v, rows_v, bag_v, sem):
        wid = lax.axis_index("s") * 2 + lax.axis_index("c")
        idx_base = wid * (BAGS_PER_W * L)
        pltpu.sync_copy(x_hbm.at[pl.ds(idx_base, BAGS_PER_W * L)], idx_v)

        @pl.loop(0, N_CHUNKS)
        def _chunk(ci):
            pltpu.async_copy(
                emb_hbm.at[idx_v.at[pl.ds(ci * CHUNK_IDX, CHUNK_IDX)]], rows_v, sem
            ).wait()

            @pl.loop(0, CHUNK_BAGS)
            def _bag(bi):
                base = bi * L
                accs = [rows_v[base, pl.ds(16 * c, 16)] for c in range(4)]
                for l in range(1, L):
                    for c in range(4):
                        accs[c] = accs[c] + rows_v[base + l, pl.ds(16 * c, 16)]
                out_row = ci * CHUNK_BAGS + bi
                for c in range(4):
                    bag_v[out_row, pl.ds(16 * c, 16)] = accs[c] * (1.0 / L)

        pltpu.sync_copy(bag_v, out_hbm.at[pl.ds(wid * BAGS_PER_W, BAGS_PER_W)])

    return _bag_mean_sc


MLP_BLK = 1024


def _mlp_body(bag_ref, w1_ref, b1_ref, w2_ref, b2_ref, w3_ref, b3_ref, out_ref):
    x = bag_ref[...]
    h = jnp.dot(x, w1_ref[...], preferred_element_type=jnp.float32) + b1_ref[...]
    h = jnp.maximum(h, 0.0)
    h = jnp.dot(h, w2_ref[...], preferred_element_type=jnp.float32) + b2_ref[...]
    h = jnp.maximum(h, 0.0)
    out_ref[...] = (
        jnp.dot(h, w3_ref[...], preferred_element_type=jnp.float32) + b3_ref[...]
    )


def _mlp(bag, W1, b1, W2, b2, W3, b3):
    return pl.pallas_call(
        _mlp_body,
        grid=(B // MLP_BLK,),
        in_specs=[
            pl.BlockSpec((MLP_BLK, EMBED), lambda i: (i, 0)),
            pl.BlockSpec((EMBED, 128), lambda i: (0, 0)),
            pl.BlockSpec((1, 128), lambda i: (0, 0)),
            pl.BlockSpec((128, EMBED), lambda i: (0, 0)),
            pl.BlockSpec((1, EMBED), lambda i: (0, 0)),
            pl.BlockSpec((EMBED, N_CLASSES), lambda i: (0, 0)),
            pl.BlockSpec((1, N_CLASSES), lambda i: (0, 0)),
        ],
        out_specs=pl.BlockSpec((MLP_BLK, N_CLASSES), lambda i: (i, 0)),
        out_shape=jax.ShapeDtypeStruct((B, N_CLASSES), jnp.float32),
    )(bag, W1, b1, W2, b2, W3, b3)


def kernel(X_batch, emb, W1, b1, W2, b2, W3, b3):
    x_flat = X_batch.astype(jnp.int32).reshape(-1)
    bag = _get_bag_mean_sc()(x_flat, emb)
    return _mlp(
        bag,
        W1,
        b1.reshape(1, -1),
        W2,
        b2.reshape(1, -1),
        W3,
        b3.reshape(1, -1),
    )


# speedup vs baseline: 5.6661x; 1.2452x over previous
import functools

import jax
import jax.numpy as jnp
from jax import lax
from jax.experimental import pallas as pl
from jax.experimental.pallas import tpu as pltpu
from jax.experimental.pallas import tpu_sc as plsc

VOCAB = 100000
EMBED = 64
N_CLASSES = 1000
B = 16384
L = 20

NW = 32
BAGS_PER_W = B // NW
CHUNK_BAGS = 4
CHUNK_IDX = CHUNK_BAGS * L
N_CHUNKS = BAGS_PER_W // CHUNK_BAGS

@functools.cache
def _get_bag_mean_sc():
    mesh = plsc.VectorSubcoreMesh(core_axis_name="c", subcore_axis_name="s")

    @functools.partial(
        pl.kernel,
        out_type=jax.ShapeDtypeStruct((B, EMBED), jnp.float32),
        mesh=mesh,
        scratch_types=[
            pltpu.VMEM((BAGS_PER_W * L,), jnp.int32),
            pltpu.VMEM((CHUNK_IDX, EMBED), jnp.float32),
            pltpu.VMEM((CHUNK_IDX, EMBED), jnp.float32),
            pltpu.VMEM((BAGS_PER_W, EMBED), jnp.float32),
            pltpu.SemaphoreType.DMA,
            pltpu.SemaphoreType.DMA,
        ],
        compiler_params=pltpu.CompilerParams(use_tc_tiling_on_sc=False),
    )
    def _bag_mean_sc(x_hbm, emb_hbm, out_hbm, idx_v, rows0_v, rows1_v, bag_v,
                     sem0, sem1):
        wid = lax.axis_index("s") * 2 + lax.axis_index("c")
        idx_base = wid * (BAGS_PER_W * L)
        pltpu.sync_copy(x_hbm.at[pl.ds(idx_base, BAGS_PER_W * L)], idx_v)

        bufs = (rows0_v, rows1_v)
        sems = (sem0, sem1)

        def _gather(ci, buf, sem):
            return pltpu.async_copy(
                emb_hbm.at[idx_v.at[pl.ds(ci * CHUNK_IDX, CHUNK_IDX)]], buf, sem
            )

        _gather(0, bufs[0], sems[0])

        def _compute(ci, buf):
            @pl.loop(0, CHUNK_BAGS)
            def _bag(bi):
                base = bi * L
                accs = [buf[base, pl.ds(16 * c, 16)] for c in range(4)]
                for l in range(1, L):
                    for c in range(4):
                        accs[c] = accs[c] + buf[base + l, pl.ds(16 * c, 16)]
                out_row = ci * CHUNK_BAGS + bi
                for c in range(4):
                    bag_v[out_row, pl.ds(16 * c, 16)] = accs[c] * (1.0 / L)

        @pl.loop(0, N_CHUNKS // 2)
        def _chunk(ci2):
            for parity in (0, 1):
                ci = ci2 * 2 + parity
                nxt = 1 - parity

                @pl.when(ci + 1 < N_CHUNKS)
                def _():
                    _gather(ci + 1, bufs[nxt], sems[nxt])

                pltpu.make_async_copy(
                    emb_hbm.at[idx_v.at[pl.ds(ci * CHUNK_IDX, CHUNK_IDX)]],
                    bufs[parity],
                    sems[parity],
                ).wait()
                _compute(ci, bufs[parity])

        pltpu.sync_copy(bag_v, out_hbm.at[pl.ds(wid * BAGS_PER_W, BAGS_PER_W)])

    return _bag_mean_sc


MLP_BLK = 1024


def _mlp_body(bag_ref, w1_ref, b1_ref, w2_ref, b2_ref, w3_ref, b3_ref, out_ref):
    x = bag_ref[...]
    h = jnp.dot(x, w1_ref[...], preferred_element_type=jnp.float32) + b1_ref[...]
    h = jnp.maximum(h, 0.0)
    h = jnp.dot(h, w2_ref[...], preferred_element_type=jnp.float32) + b2_ref[...]
    h = jnp.maximum(h, 0.0)
    out_ref[...] = (
        jnp.dot(h, w3_ref[...], preferred_element_type=jnp.float32) + b3_ref[...]
    )


def _mlp(bag, W1, b1, W2, b2, W3, b3):
    return pl.pallas_call(
        _mlp_body,
        grid=(B // MLP_BLK,),
        in_specs=[
            pl.BlockSpec((MLP_BLK, EMBED), lambda i: (i, 0)),
            pl.BlockSpec((EMBED, 128), lambda i: (0, 0)),
            pl.BlockSpec((1, 128), lambda i: (0, 0)),
            pl.BlockSpec((128, EMBED), lambda i: (0, 0)),
            pl.BlockSpec((1, EMBED), lambda i: (0, 0)),
            pl.BlockSpec((EMBED, N_CLASSES), lambda i: (0, 0)),
            pl.BlockSpec((1, N_CLASSES), lambda i: (0, 0)),
        ],
        out_specs=pl.BlockSpec((MLP_BLK, N_CLASSES), lambda i: (i, 0)),
        out_shape=jax.ShapeDtypeStruct((B, N_CLASSES), jnp.float32),
    )(bag, W1, b1, W2, b2, W3, b3)


def kernel(X_batch, emb, W1, b1, W2, b2, W3, b3):
    x_flat = X_batch.astype(jnp.int32).reshape(-1)
    bag = _get_bag_mean_sc()(x_flat, emb)
    return _mlp(
        bag,
        W1,
        b1.reshape(1, -1),
        W2,
        b2.reshape(1, -1),
        W3,
        b3.reshape(1, -1),
    )


# speedup vs baseline: 7.3251x; 1.2928x over previous
import functools

import jax
import jax.numpy as jnp
from jax import lax
from jax.experimental import pallas as pl
from jax.experimental.pallas import tpu as pltpu
from jax.experimental.pallas import tpu_sc as plsc

VOCAB = 100000
EMBED = 64
N_CLASSES = 1000
B = 16384
L = 20

NW = 32
BAGS_PER_W = B // NW
CHUNK_BAGS = 4
CHUNK_IDX = CHUNK_BAGS * L
N_CHUNKS = BAGS_PER_W // CHUNK_BAGS

@functools.cache
def _get_bag_mean_sc():
    mesh = plsc.VectorSubcoreMesh(core_axis_name="c", subcore_axis_name="s")

    @functools.partial(
        pl.kernel,
        out_type=jax.ShapeDtypeStruct((B, EMBED), jnp.float32),
        mesh=mesh,
        scratch_types=[
            pltpu.VMEM((BAGS_PER_W * L,), jnp.int32),
            pltpu.VMEM((CHUNK_IDX, EMBED), jnp.float32),
            pltpu.VMEM((CHUNK_IDX, EMBED), jnp.float32),
            pltpu.VMEM((BAGS_PER_W, EMBED), jnp.float32),
            pltpu.SemaphoreType.DMA,
            pltpu.SemaphoreType.DMA,
        ],
        compiler_params=pltpu.CompilerParams(use_tc_tiling_on_sc=False),
    )
    def _bag_mean_sc(x_hbm, emb_hbm, out_hbm, idx_v, rows0_v, rows1_v, bag_v,
                     sem0, sem1):
        wid = lax.axis_index("s") * 2 + lax.axis_index("c")
        idx_base = wid * (BAGS_PER_W * L)
        pltpu.sync_copy(x_hbm.at[pl.ds(idx_base, BAGS_PER_W * L)], idx_v)

        bufs = (rows0_v, rows1_v)
        sems = (sem0, sem1)

        def _gather(ci, buf, sem):
            return pltpu.async_copy(
                emb_hbm.at[idx_v.at[pl.ds(ci * CHUNK_IDX, CHUNK_IDX)]], buf, sem
            )

        _gather(0, bufs[0], sems[0])

        def _compute(ci, buf):
            @pl.loop(0, CHUNK_BAGS)
            def _bag(bi):
                base = bi * L
                accs = [buf[base, pl.ds(16 * c, 16)] for c in range(4)]
                for l in range(1, L):
                    for c in range(4):
                        accs[c] = accs[c] + buf[base + l, pl.ds(16 * c, 16)]
                out_row = ci * CHUNK_BAGS + bi
                for c in range(4):
                    bag_v[out_row, pl.ds(16 * c, 16)] = accs[c] * (1.0 / L)

        @pl.loop(0, N_CHUNKS // 2)
        def _chunk(ci2):
            for parity in (0, 1):
                ci = ci2 * 2 + parity
                nxt = 1 - parity

                @pl.when(ci + 1 < N_CHUNKS)
                def _():
                    _gather(ci + 1, bufs[nxt], sems[nxt])

                pltpu.make_async_copy(
                    emb_hbm.at[idx_v.at[pl.ds(ci * CHUNK_IDX, CHUNK_IDX)]],
                    bufs[parity],
                    sems[parity],
                ).wait()
                _compute(ci, bufs[parity])

        pltpu.sync_copy(bag_v, out_hbm.at[pl.ds(wid * BAGS_PER_W, BAGS_PER_W)])

    return _bag_mean_sc


MLP_BLK = 2048


def _mlp_t_body(xt_ref, w1t_ref, b1t_ref, w2t_ref, b2t_ref, w3t_ref, b3t_ref,
                outt_ref):
    bf = jnp.bfloat16
    xt = xt_ref[...].astype(bf)
    h = jnp.dot(w1t_ref[...].astype(bf), xt, preferred_element_type=jnp.float32)
    h = jnp.maximum(h + b1t_ref[...], 0.0)
    h = jnp.dot(w2t_ref[...].astype(bf), h.astype(bf),
                preferred_element_type=jnp.float32)
    h = jnp.maximum(h + b2t_ref[...], 0.0)
    outt_ref[...] = (
        jnp.dot(w3t_ref[...].astype(bf), h.astype(bf),
                preferred_element_type=jnp.float32)
        + b3t_ref[...]
    )


def _mlp_t(bag_t, W1t, b1t, W2t, b2t, W3t, b3t):
    return pl.pallas_call(
        _mlp_t_body,
        grid=(B // MLP_BLK,),
        in_specs=[
            pl.BlockSpec((EMBED, MLP_BLK), lambda i: (0, i)),
            pl.BlockSpec((128, EMBED), lambda i: (0, 0)),
            pl.BlockSpec((128, 1), lambda i: (0, 0)),
            pl.BlockSpec((EMBED, 128), lambda i: (0, 0)),
            pl.BlockSpec((EMBED, 1), lambda i: (0, 0)),
            pl.BlockSpec((N_CLASSES, EMBED), lambda i: (0, 0)),
            pl.BlockSpec((N_CLASSES, 1), lambda i: (0, 0)),
        ],
        out_specs=pl.BlockSpec((N_CLASSES, MLP_BLK), lambda i: (0, i)),
        out_shape=jax.ShapeDtypeStruct((N_CLASSES, B), jnp.float32),
    )(bag_t, W1t, b1t, W2t, b2t, W3t, b3t)


def kernel(X_batch, emb, W1, b1, W2, b2, W3, b3):
    x_flat = X_batch.astype(jnp.int32).reshape(-1)
    bag = _get_bag_mean_sc()(x_flat, emb)
    out_t = _mlp_t(
        bag.T,
        W1.T,
        b1.reshape(-1, 1),
        W2.T,
        b2.reshape(-1, 1),
        W3.T,
        b3.reshape(-1, 1),
    )
    return out_t.T


# speedup vs baseline: 7.6436x; 1.0435x over previous
import functools

import jax
import jax.numpy as jnp
from jax import lax
from jax.experimental import pallas as pl
from jax.experimental.pallas import tpu as pltpu
from jax.experimental.pallas import tpu_sc as plsc

VOCAB = 100000
EMBED = 64
N_CLASSES = 1000
B = 16384
L = 20

NW = 32
BAGS_PER_W = B // NW
CHUNK_BAGS = 4
CHUNK_IDX = CHUNK_BAGS * L
N_CHUNKS = BAGS_PER_W // CHUNK_BAGS

@functools.cache
def _get_bag_mean_sc():
    mesh = plsc.VectorSubcoreMesh(core_axis_name="c", subcore_axis_name="s")

    @functools.partial(
        pl.kernel,
        out_type=jax.ShapeDtypeStruct((B, EMBED), jnp.float32),
        mesh=mesh,
        scratch_types=[
            pltpu.VMEM((BAGS_PER_W * L,), jnp.int32),
            pltpu.VMEM((CHUNK_IDX, EMBED), jnp.float32),
            pltpu.VMEM((CHUNK_IDX, EMBED), jnp.float32),
            pltpu.VMEM((BAGS_PER_W, EMBED), jnp.float32),
            pltpu.SemaphoreType.DMA,
            pltpu.SemaphoreType.DMA,
        ],
        compiler_params=pltpu.CompilerParams(use_tc_tiling_on_sc=False),
    )
    def _bag_mean_sc(x_hbm, emb_hbm, out_hbm, idx_v, rows0_v, rows1_v, bag_v,
                     sem0, sem1):
        wid = lax.axis_index("s") * 2 + lax.axis_index("c")
        idx_base = wid * (BAGS_PER_W * L)
        pltpu.sync_copy(x_hbm.at[pl.ds(idx_base, BAGS_PER_W * L)], idx_v)

        bufs = (rows0_v, rows1_v)
        sems = (sem0, sem1)

        def _gather(ci, buf, sem):
            return pltpu.async_copy(
                emb_hbm.at[idx_v.at[pl.ds(ci * CHUNK_IDX, CHUNK_IDX)]], buf, sem
            )

        _gather(0, bufs[0], sems[0])

        def _compute(ci, buf):
            @pl.loop(0, CHUNK_BAGS)
            def _bag(bi):
                base = bi * L
                accs = [buf[base, pl.ds(16 * c, 16)] for c in range(4)]
                for l in range(1, L):
                    for c in range(4):
                        accs[c] = accs[c] + buf[base + l, pl.ds(16 * c, 16)]
                out_row = ci * CHUNK_BAGS + bi
                for c in range(4):
                    bag_v[out_row, pl.ds(16 * c, 16)] = accs[c] * (1.0 / L)

        @pl.loop(0, N_CHUNKS // 2)
        def _chunk(ci2):
            for parity in (0, 1):
                ci = ci2 * 2 + parity
                nxt = 1 - parity

                @pl.when(ci + 1 < N_CHUNKS)
                def _():
                    _gather(ci + 1, bufs[nxt], sems[nxt])

                pltpu.make_async_copy(
                    emb_hbm.at[idx_v.at[pl.ds(ci * CHUNK_IDX, CHUNK_IDX)]],
                    bufs[parity],
                    sems[parity],
                ).wait()
                _compute(ci, bufs[parity])

        pltpu.sync_copy(bag_v, out_hbm.at[pl.ds(wid * BAGS_PER_W, BAGS_PER_W)])

    return _bag_mean_sc


MLP_BLK = 2048


def _mlp_t_body(xt_ref, w1t_ref, b1t_ref, w2t_ref, b2t_ref, w3t_ref, b3t_ref,
                outt_ref):
    bf = jnp.bfloat16
    xt = xt_ref[...].astype(bf)
    h = jnp.dot(w1t_ref[...].astype(bf), xt, preferred_element_type=jnp.float32)
    h = jnp.maximum(h + b1t_ref[...], 0.0)
    h = jnp.dot(w2t_ref[...].astype(bf), h.astype(bf),
                preferred_element_type=jnp.float32)
    h = jnp.maximum(h + b2t_ref[...], 0.0)
    outt_ref[...] = (
        jnp.dot(w3t_ref[...].astype(bf), h.astype(bf),
                preferred_element_type=jnp.float32)
        + b3t_ref[...]
    )


def _mlp_t(bag_t, W1t, b1t, W2t, b2t, W3t, b3t):
    return pl.pallas_call(
        _mlp_t_body,
        grid=(B // MLP_BLK,),
        in_specs=[
            pl.BlockSpec((EMBED, MLP_BLK), lambda i: (0, i)),
            pl.BlockSpec((128, EMBED), lambda i: (0, 0)),
            pl.BlockSpec((128, 1), lambda i: (0, 0)),
            pl.BlockSpec((EMBED, 128), lambda i: (0, 0)),
            pl.BlockSpec((EMBED, 1), lambda i: (0, 0)),
            pl.BlockSpec((N_CLASSES, EMBED), lambda i: (0, 0)),
            pl.BlockSpec((N_CLASSES, 1), lambda i: (0, 0)),
        ],
        out_specs=pl.BlockSpec((N_CLASSES, MLP_BLK), lambda i: (0, i)),
        out_shape=jax.ShapeDtypeStruct((N_CLASSES, B), jnp.float32),
    )(bag_t, W1t, b1t, W2t, b2t, W3t, b3t)


def kernel(X_batch, emb, W1, b1, W2, b2, W3, b3):
    x_flat = X_batch.astype(jnp.int32).reshape(-1) * 2
    emb_pad = jnp.pad(emb, ((0, 0), (0, EMBED))).reshape(2 * VOCAB, EMBED)
    bag = _get_bag_mean_sc()(x_flat, emb_pad)
    out_t = _mlp_t(
        bag.T,
        W1.T,
        b1.reshape(-1, 1),
        W2.T,
        b2.reshape(-1, 1),
        W3.T,
        b3.reshape(-1, 1),
    )
    return out_t.T
